# baseline (speedup 1.0000x reference)
import functools
import math

import numpy as np
import jax
import jax.numpy as jnp
from jax import lax
from jax.experimental import pallas as pl
from jax.experimental.pallas import tpu as pltpu
from jax.experimental.pallas import tpu_sc as plsc

D_MODEL = 768
SCALE = math.sqrt(768.0)
NW = 32
CHUNK = 32


def _pos_encoding(seq_len: int, d: int) -> np.ndarray:
    position = np.arange(seq_len, dtype=np.float32)
    num_timescales = d // 2
    log_inc = math.log(10000.0) / max(1, num_timescales - 1)
    inv = np.exp(np.arange(num_timescales, dtype=np.float32) * np.float32(-log_inc))
    scaled = position[:, None] * inv[None, :].astype(np.float32)
    pe = np.zeros((seq_len, d), np.float32)
    pe[:, 0::2] = np.sin(scaled)
    pe[:, 1::2] = np.cos(scaled)
    return pe


def kernel(x, table):
    bsz, seq = x.shape
    d = table.shape[1]
    B = bsz * seq
    s_per_w = seq // NW
    b_per_w = bsz * s_per_w
    nch = b_per_w // CHUNK
    ch_per_b = s_per_w // CHUNK
    nvec = d // 16

    pos = jnp.asarray(_pos_encoding(seq, d))

    mesh = plsc.VectorSubcoreMesh(core_axis_name="c", subcore_axis_name="s")

    @functools.partial(
        pl.kernel,
        mesh=mesh,
        out_type=jax.ShapeDtypeStruct((B, d), jnp.float32),
        scratch_types=[
            pltpu.VMEM((b_per_w,), jnp.int32),
            pltpu.VMEM((s_per_w, d), jnp.float32),
            pltpu.VMEM((2, CHUNK, d), jnp.float32),
            pltpu.SemaphoreType.DMA,
            pltpu.SemaphoreType.DMA,
            pltpu.SemaphoreType.DMA,
            pltpu.SemaphoreType.DMA,
            pltpu.SemaphoreType.DMA,
        ],
    )
    def emb_kernel(x_hbm, pos_hbm, table_hbm, out_hbm,
                   idx_v, posbuf, gbuf, g0, g1, o0, o1, psem):
        gsem = (g0, g1)
        osem = (o0, o1)
        wid = lax.axis_index("s") * 2 + lax.axis_index("c")
        spos = wid * s_per_w
        hpos = pltpu.async_copy(pos_hbm.at[pl.ds(spos, s_per_w)], posbuf, psem)
        for b in range(bsz):
            pltpu.sync_copy(x_hbm.at[b, pl.ds(spos, s_per_w)],
                            idx_v.at[pl.ds(b * s_per_w, s_per_w)])

        def start_chunk(k):
            slot = k % 2
            return pltpu.async_copy(
                table_hbm.at[idx_v.at[pl.ds(k * CHUNK, CHUNK)]], gbuf.at[slot],
                gsem[slot])

        hg = [None, None]
        ho = [None, None]
        hg[0] = start_chunk(0)
        hpos.wait()
        for k in range(nch):
            slot = k % 2
            nxt = (k + 1) % 2
            if k + 1 < nch:
                if ho[nxt] is not None:
                    ho[nxt].wait()
                    ho[nxt] = None
                hg[nxt] = start_chunk(k + 1)
            hg[slot].wait()
            p0 = (k % ch_per_b) * CHUNK
            gb = gbuf.at[slot]

            @plsc.parallel_loop(0, CHUNK, 1, unroll=4)
            def row_body(r):
                for j in range(nvec):
                    g = gb[r, pl.ds(j * 16, 16)]
                    p = posbuf[p0 + r, pl.ds(j * 16, 16)]
                    gb[r, pl.ds(j * 16, 16)] = g * SCALE + p

            out0 = (k // ch_per_b) * seq + spos + p0
            ho[slot] = pltpu.async_copy(
                gb, out_hbm.at[pl.ds(out0, CHUNK)], osem[slot])
        for h in ho:
            if h is not None:
                h.wait()

    out = emb_kernel(x, pos, table)
    return out.reshape(bsz, seq, d)


# speedup vs baseline: 1.0562x; 1.0562x over previous
import functools
import math

import numpy as np
import jax
import jax.numpy as jnp
from jax import lax
from jax.experimental import pallas as pl
from jax.experimental.pallas import tpu as pltpu
from jax.experimental.pallas import tpu_sc as plsc

D_MODEL = 768
SCALE = math.sqrt(768.0)
NW = 32
CHUNK = 32


def _pos_encoding(seq_len: int, d: int) -> np.ndarray:
    position = np.arange(seq_len, dtype=np.float32)
    num_timescales = d // 2
    log_inc = math.log(10000.0) / max(1, num_timescales - 1)
    inv = np.exp(np.arange(num_timescales, dtype=np.float32) * np.float32(-log_inc))
    scaled = position[:, None] * inv[None, :].astype(np.float32)
    pe = np.zeros((seq_len, d), np.float32)
    pe[:, 0::2] = np.sin(scaled)
    pe[:, 1::2] = np.cos(scaled)
    return pe


def kernel(x, table):
    bsz, seq = x.shape
    d = table.shape[1]
    B = bsz * seq
    s_per_w = seq // NW
    b_per_w = bsz * s_per_w
    nch = b_per_w // CHUNK
    ch_per_b = s_per_w // CHUNK
    nvec = d // 16

    pos = jnp.asarray(_pos_encoding(seq, d))

    mesh = plsc.VectorSubcoreMesh(core_axis_name="c", subcore_axis_name="s")

    @functools.partial(
        pl.kernel,
        mesh=mesh,
        out_type=jax.ShapeDtypeStruct((B, d), jnp.float32),
        scratch_types=[
            pltpu.VMEM((b_per_w,), jnp.int32),
            pltpu.VMEM((s_per_w, d), jnp.float32),
            pltpu.VMEM((2, CHUNK, d), jnp.float32),
            pltpu.SemaphoreType.DMA,
            pltpu.SemaphoreType.DMA,
            pltpu.SemaphoreType.DMA,
            pltpu.SemaphoreType.DMA,
            pltpu.SemaphoreType.DMA,
        ],
    )
    def emb_kernel(x_hbm, pos_hbm, table_hbm, out_hbm,
                   idx_v, posbuf, gbuf, g0, g1, o0, o1, psem):
        gsem = (g0, g1)
        osem = (o0, o1)
        wid = lax.axis_index("s") * 2 + lax.axis_index("c")
        spos = wid * s_per_w
        hpos = pltpu.async_copy(pos_hbm.at[pl.ds(spos, s_per_w)], posbuf, psem)
        for b in range(bsz):
            pltpu.sync_copy(x_hbm.at[b, pl.ds(spos, s_per_w)],
                            idx_v.at[pl.ds(b * s_per_w, s_per_w)])

        def start_chunk(k):
            slot = k % 2
            return pltpu.async_copy(
                table_hbm.at[idx_v.at[pl.ds(k * CHUNK, CHUNK)]], gbuf.at[slot],
                gsem[slot])

        hg = [None, None]
        ho = [None, None]
        hg[0] = start_chunk(0)
        hpos.wait()
        for k in range(nch):
            slot = k % 2
            nxt = (k + 1) % 2
            if k + 1 < nch:
                if ho[nxt] is not None:
                    ho[nxt].wait()
                    ho[nxt] = None
                hg[nxt] = start_chunk(k + 1)
            hg[slot].wait()
            p0 = (k % ch_per_b) * CHUNK
            gb = gbuf.at[slot]

            @plsc.parallel_loop(0, CHUNK, 1, unroll=2)
            def row_body(r):
                for j in range(nvec):
                    g = gb[r, pl.ds(j * 16, 16)]
                    p = posbuf[p0 + r, pl.ds(j * 16, 16)]
                    gb[r, pl.ds(j * 16, 16)] = g * SCALE + p

            out0 = (k // ch_per_b) * seq + spos + p0
            ho[slot] = pltpu.async_copy(
                gb, out_hbm.at[pl.ds(out0, CHUNK)], osem[slot])
        for h in ho:
            if h is not None:
                h.wait()

    out = emb_kernel(x, pos, table)
    return out.reshape(bsz, seq, d)


# speedup vs baseline: 1.0809x; 1.0234x over previous
import functools
import math

import numpy as np
import jax
import jax.numpy as jnp
from jax import lax
from jax.experimental import pallas as pl
from jax.experimental.pallas import tpu as pltpu
from jax.experimental.pallas import tpu_sc as plsc

D_MODEL = 768
SCALE = math.sqrt(768.0)
NW = 32
CHUNK = 32


def _pos_encoding(seq_len: int, d: int) -> np.ndarray:
    position = np.arange(seq_len, dtype=np.float32)
    num_timescales = d // 2
    log_inc = math.log(10000.0) / max(1, num_timescales - 1)
    inv = np.exp(np.arange(num_timescales, dtype=np.float32) * np.float32(-log_inc))
    scaled = position[:, None] * inv[None, :].astype(np.float32)
    pe = np.zeros((seq_len, d), np.float32)
    pe[:, 0::2] = np.sin(scaled)
    pe[:, 1::2] = np.cos(scaled)
    return pe


def kernel(x, table):
    bsz, seq = x.shape
    d = table.shape[1]
    B = bsz * seq
    s_per_w = seq // NW
    b_per_w = bsz * s_per_w
    nch = b_per_w // CHUNK
    ch_per_b = s_per_w // CHUNK
    nvec = d // 16

    pe = _pos_encoding(seq, d)
    bits = pe.view(np.uint32).reshape(seq, d // 32, 2, 16)
    words = (bits[:, :, 0, :] >> 16) | (bits[:, :, 1, :] & np.uint32(0xFFFF0000))
    pos = jnp.asarray(words.reshape(-1).view(np.int32))

    mesh = plsc.VectorSubcoreMesh(core_axis_name="c", subcore_axis_name="s")

    @functools.partial(
        pl.kernel,
        mesh=mesh,
        out_type=jax.ShapeDtypeStruct((B, d), jnp.float32),
        scratch_types=[
            pltpu.VMEM((b_per_w,), jnp.int32),
            pltpu.VMEM((s_per_w * d // 2,), jnp.int32),
            pltpu.VMEM((2, CHUNK, d), jnp.float32),
            pltpu.SemaphoreType.DMA,
            pltpu.SemaphoreType.DMA,
            pltpu.SemaphoreType.DMA,
            pltpu.SemaphoreType.DMA,
            pltpu.SemaphoreType.DMA,
        ],
    )
    def emb_kernel(x_hbm, pos_hbm, table_hbm, out_hbm,
                   idx_v, posbuf, gbuf, g0, g1, o0, o1, psem):
        gsem = (g0, g1)
        osem = (o0, o1)
        wid = lax.axis_index("s") * 2 + lax.axis_index("c")
        spos = wid * s_per_w
        hpos = pltpu.async_copy(
            pos_hbm.at[pl.ds(pl.multiple_of(spos * (d // 2), 8),
                             s_per_w * d // 2)], posbuf, psem)
        for b in range(bsz):
            pltpu.sync_copy(x_hbm.at[b, pl.ds(spos, s_per_w)],
                            idx_v.at[pl.ds(b * s_per_w, s_per_w)])

        def start_chunk(k):
            slot = k % 2
            return pltpu.async_copy(
                table_hbm.at[idx_v.at[pl.ds(k * CHUNK, CHUNK)]], gbuf.at[slot],
                gsem[slot])

        hg = [None, None]
        ho = [None, None]
        hg[0] = start_chunk(0)
        hpos.wait()
        for k in range(nch):
            slot = k % 2
            nxt = (k + 1) % 2
            if k + 1 < nch:
                if ho[nxt] is not None:
                    ho[nxt].wait()
                    ho[nxt] = None
                hg[nxt] = start_chunk(k + 1)
            hg[slot].wait()
            p0 = (k % ch_per_b) * CHUNK
            gb = gbuf.at[slot]

            @plsc.parallel_loop(0, CHUNK, 1, unroll=2)
            def row_body(r):
                for t in range(nvec // 2):
                    off = pl.multiple_of((p0 + r) * (d // 2) + t * 16, 8)
                    w = posbuf[pl.ds(off, 16)]
                    pa = lax.bitcast_convert_type(w << 16, jnp.float32)
                    pb = lax.bitcast_convert_type(w, jnp.float32)
                    g0 = gb[r, pl.ds(t * 32, 16)]
                    gb[r, pl.ds(t * 32, 16)] = g0 * SCALE + pa
                    g1 = gb[r, pl.ds(t * 32 + 16, 16)]
                    gb[r, pl.ds(t * 32 + 16, 16)] = g1 * SCALE + pb

            out0 = (k // ch_per_b) * seq + spos + p0
            ho[slot] = pltpu.async_copy(
                gb, out_hbm.at[pl.ds(out0, CHUNK)], osem[slot])
        for h in ho:
            if h is not None:
                h.wait()

    out = emb_kernel(x, pos, table)
    return out.reshape(bsz, seq, d)


# speedup vs baseline: 1.1811x; 1.0927x over previous
import functools
import math

import numpy as np
import jax
import jax.numpy as jnp
from jax import lax
from jax.experimental import pallas as pl
from jax.experimental.pallas import tpu as pltpu
from jax.experimental.pallas import tpu_sc as plsc

D_MODEL = 768
SCALE = math.sqrt(768.0)
NW = 32
CHUNK = 64


def _pos_encoding(seq_len: int, d: int) -> np.ndarray:
    position = np.arange(seq_len, dtype=np.float32)
    num_timescales = d // 2
    log_inc = math.log(10000.0) / max(1, num_timescales - 1)
    inv = np.exp(np.arange(num_timescales, dtype=np.float32) * np.float32(-log_inc))
    scaled = position[:, None] * inv[None, :].astype(np.float32)
    pe = np.zeros((seq_len, d), np.float32)
    pe[:, 0::2] = np.sin(scaled)
    pe[:, 1::2] = np.cos(scaled)
    return pe


def kernel(x, table):
    bsz, seq = x.shape
    d = table.shape[1]
    B = bsz * seq
    s_per_w = seq // NW
    b_per_w = bsz * s_per_w
    nch = b_per_w // CHUNK
    ch_per_b = s_per_w // CHUNK
    nvec = d // 16

    pe = _pos_encoding(seq, d)
    bits = pe.view(np.uint32).reshape(seq, d // 32, 2, 16)
    words = (bits[:, :, 0, :] >> 16) | (bits[:, :, 1, :] & np.uint32(0xFFFF0000))
    pos = jnp.asarray(words.reshape(-1).view(np.int32))

    mesh = plsc.VectorSubcoreMesh(core_axis_name="c", subcore_axis_name="s")

    @functools.partial(
        pl.kernel,
        mesh=mesh,
        out_type=jax.ShapeDtypeStruct((B, d), jnp.float32),
        scratch_types=[
            pltpu.VMEM((b_per_w,), jnp.int32),
            pltpu.VMEM((s_per_w * d // 2,), jnp.int32),
            pltpu.VMEM((2, CHUNK, d), jnp.float32),
            pltpu.SemaphoreType.DMA,
            pltpu.SemaphoreType.DMA,
            pltpu.SemaphoreType.DMA,
            pltpu.SemaphoreType.DMA,
            pltpu.SemaphoreType.DMA,
        ],
    )
    def emb_kernel(x_hbm, pos_hbm, table_hbm, out_hbm,
                   idx_v, posbuf, gbuf, g0, g1, o0, o1, psem):
        gsem = (g0, g1)
        osem = (o0, o1)
        wid = lax.axis_index("s") * 2 + lax.axis_index("c")
        spos = wid * s_per_w
        hpos = pltpu.async_copy(
            pos_hbm.at[pl.ds(pl.multiple_of(spos * (d // 2), 8),
                             s_per_w * d // 2)], posbuf, psem)
        for b in range(bsz):
            pltpu.sync_copy(x_hbm.at[b, pl.ds(spos, s_per_w)],
                            idx_v.at[pl.ds(b * s_per_w, s_per_w)])

        def start_chunk(k):
            slot = k % 2
            return pltpu.async_copy(
                table_hbm.at[idx_v.at[pl.ds(k * CHUNK, CHUNK)]], gbuf.at[slot],
                gsem[slot])

        hg = [None, None]
        ho = [None, None]
        hg[0] = start_chunk(0)
        hpos.wait()
        for k in range(nch):
            slot = k % 2
            nxt = (k + 1) % 2
            if k + 1 < nch:
                if ho[nxt] is not None:
                    ho[nxt].wait()
                    ho[nxt] = None
                hg[nxt] = start_chunk(k + 1)
            hg[slot].wait()
            p0 = (k % ch_per_b) * CHUNK
            gb = gbuf.at[slot]

            @plsc.parallel_loop(0, CHUNK, 1, unroll=2)
            def row_body(r):
                for t in range(nvec // 2):
                    off = pl.multiple_of((p0 + r) * (d // 2) + t * 16, 8)
                    w = posbuf[pl.ds(off, 16)]
                    pa = lax.bitcast_convert_type(w << 16, jnp.float32)
                    pb = lax.bitcast_convert_type(w, jnp.float32)
                    g0 = gb[r, pl.ds(t * 32, 16)]
                    gb[r, pl.ds(t * 32, 16)] = g0 * SCALE + pa
                    g1 = gb[r, pl.ds(t * 32 + 16, 16)]
                    gb[r, pl.ds(t * 32 + 16, 16)] = g1 * SCALE + pb

            out0 = (k // ch_per_b) * seq + spos + p0
            ho[slot] = pltpu.async_copy(
                gb, out_hbm.at[pl.ds(out0, CHUNK)], osem[slot])
        for h in ho:
            if h is not None:
                h.wait()

    out = emb_kernel(x, pos, table)
    return out.reshape(bsz, seq, d)


# speedup vs baseline: 1.2399x; 1.0498x over previous
import functools
import math

import numpy as np
import jax
import jax.numpy as jnp
from jax import lax
from jax.experimental import pallas as pl
from jax.experimental.pallas import tpu as pltpu
from jax.experimental.pallas import tpu_sc as plsc

D_MODEL = 768
SCALE = math.sqrt(768.0)
NW = 32
CHUNK = 64


def _pos_encoding(seq_len: int, d: int) -> np.ndarray:
    position = np.arange(seq_len, dtype=np.float32)
    num_timescales = d // 2
    log_inc = math.log(10000.0) / max(1, num_timescales - 1)
    inv = np.exp(np.arange(num_timescales, dtype=np.float32) * np.float32(-log_inc))
    scaled = position[:, None] * inv[None, :].astype(np.float32)
    pe = np.zeros((seq_len, d), np.float32)
    pe[:, 0::2] = np.sin(scaled)
    pe[:, 1::2] = np.cos(scaled)
    return pe


def kernel(x, table):
    bsz, seq = x.shape
    d = table.shape[1]
    B = bsz * seq
    s_per_w = seq // NW
    b_per_w = bsz * s_per_w
    nch = b_per_w // CHUNK
    ch_per_b = s_per_w // CHUNK
    nvec = d // 16

    pe = _pos_encoding(seq, d)
    bits = pe.view(np.uint32).reshape(seq, d // 32, 2, 16)
    words = (bits[:, :, 0, :] >> 16) | (bits[:, :, 1, :] & np.uint32(0xFFFF0000))
    pos = jnp.asarray(words.reshape(-1).view(np.int32))

    mesh = plsc.VectorSubcoreMesh(core_axis_name="c", subcore_axis_name="s")

    @functools.partial(
        pl.kernel,
        mesh=mesh,
        out_type=jax.ShapeDtypeStruct((B, d), jnp.float32),
        scratch_types=[
            pltpu.VMEM((b_per_w,), jnp.int32),
            pltpu.VMEM((s_per_w * d // 2,), jnp.int32),
            pltpu.VMEM((2, CHUNK, d), jnp.float32),
            pltpu.SemaphoreType.DMA,
            pltpu.SemaphoreType.DMA,
            pltpu.SemaphoreType.DMA,
            pltpu.SemaphoreType.DMA,
            pltpu.SemaphoreType.DMA,
            pltpu.SemaphoreType.DMA,
        ],
    )
    def emb_kernel(x_hbm, pos_hbm, table_hbm, out_hbm,
                   idx_v, posbuf, gbuf, g0, g1, o0, o1, psem, isem):
        gsem = (g0, g1)
        osem = (o0, o1)
        wid = lax.axis_index("s") * 2 + lax.axis_index("c")
        spos = wid * s_per_w
        hpos = pltpu.async_copy(
            pos_hbm.at[pl.ds(pl.multiple_of(spos * (d // 2), 8),
                             s_per_w * d // 2)], posbuf, psem)
        hidx = [
            pltpu.async_copy(x_hbm.at[b, pl.ds(spos, s_per_w)],
                             idx_v.at[pl.ds(b * s_per_w, s_per_w)], isem)
            for b in range(bsz)
        ]

        def start_chunk(k):
            slot = k % 2
            return pltpu.async_copy(
                table_hbm.at[idx_v.at[pl.ds(k * CHUNK, CHUNK)]], gbuf.at[slot],
                gsem[slot])

        hg = [None, None]
        ho = [None, None]
        for h in hidx:
            h.wait()
        hg[0] = start_chunk(0)
        hpos.wait()
        for k in range(nch):
            slot = k % 2
            nxt = (k + 1) % 2
            if k + 1 < nch:
                if ho[nxt] is not None:
                    ho[nxt].wait()
                    ho[nxt] = None
                hg[nxt] = start_chunk(k + 1)
            hg[slot].wait()
            p0 = (k % ch_per_b) * CHUNK
            gb = gbuf.at[slot]

            @plsc.parallel_loop(0, CHUNK, 1, unroll=2)
            def row_body(r):
                for t in range(nvec // 2):
                    off = pl.multiple_of((p0 + r) * (d // 2) + t * 16, 8)
                    w = posbuf[pl.ds(off, 16)]
                    pa = lax.bitcast_convert_type(w << 16, jnp.float32)
                    pb = lax.bitcast_convert_type(w, jnp.float32)
                    g0 = gb[r, pl.ds(t * 32, 16)]
                    gb[r, pl.ds(t * 32, 16)] = g0 * SCALE + pa
                    g1 = gb[r, pl.ds(t * 32 + 16, 16)]
                    gb[r, pl.ds(t * 32 + 16, 16)] = g1 * SCALE + pb

            out0 = (k // ch_per_b) * seq + spos + p0
            ho[slot] = pltpu.async_copy(
                gb, out_hbm.at[pl.ds(out0, CHUNK)], osem[slot])
        for h in ho:
            if h is not None:
                h.wait()

    out = emb_kernel(x, pos, table)
    return out.reshape(bsz, seq, d)
